# baseline (speedup 1.0000x reference)
import functools

import jax
import jax.numpy as jnp
from jax import lax
from jax.experimental import pallas as pl
from jax.experimental.pallas import tpu as pltpu
from jax.experimental.pallas import tpu_sc as plsc

_G = 52
_GG = _G * _G
_NA = 3
_NATTR = 85
_B = 32
_STRIDE = 8.0
_ANCH = ((10.0, 13.0), (16.0, 30.0), (33.0, 23.0))
_CH = 208
_NCHUNK = _GG // _CH
_NV = _CH // 16
_NC = 2
_NS = 16


def _sigmoid(v):
  return 1.0 / (1.0 + jnp.exp(-v))


def _sc_body(x_hbm, out_hbm, in_ref, out_ref):
  b = lax.axis_index("s") * _NC + lax.axis_index("c")
  lane = lax.broadcasted_iota(jnp.int32, (16,), 0)

  for a in range(_NA):
    aw = _ANCH[a][0] / _STRIDE
    ah = _ANCH[a][1] / _STRIDE

    def chunk_body(k, carry, a=a, aw=aw, ah=ah):
      pltpu.sync_copy(
          x_hbm.at[b, pl.ds(a * _NATTR, _NATTR), pl.ds(k * _CH, _CH)], in_ref)

      for j in range(_NV):
        sl = pl.ds(j * 16, 16)
        s_vec = lane + (j * 16)
        sg = k * _CH + s_vec
        q = lax.div(sg, _G)
        r = sg - q * _G
        gx = r.astype(jnp.float32)
        gy = q.astype(jnp.float32)
        r0 = (_sigmoid(in_ref[0, sl]) + gx) * _STRIDE
        r1 = (_sigmoid(in_ref[1, sl]) + gy) * _STRIDE
        r2 = (jnp.exp(in_ref[2, sl]) * aw) * _STRIDE
        r3 = (jnp.exp(in_ref[3, sl]) * ah) * _STRIDE
        for c, val in ((0, r0), (1, r1), (2, r2), (3, r3)):
          cv = jnp.full((16,), c, jnp.int32)
          plsc.store_scatter(out_ref, [s_vec, cv], val)

      def c_body(c, carry2):
        cv = jnp.zeros((16,), jnp.int32) + c
        for j in range(_NV):
          val = _sigmoid(in_ref[c, pl.ds(j * 16, 16)])
          plsc.store_scatter(out_ref, [lane + j * 16, cv], val)
        return carry2

      lax.fori_loop(4, _NATTR, c_body, 0)

      pltpu.sync_copy(
          out_ref, out_hbm.at[b, pl.ds(a * _GG + k * _CH, _CH), :])
      return carry

    lax.fori_loop(0, _NCHUNK, chunk_body, 0)


@functools.partial(
    pl.kernel,
    out_type=jax.ShapeDtypeStruct((_B, _NA * _GG, _NATTR), jnp.float32),
    mesh=plsc.VectorSubcoreMesh(core_axis_name="c", subcore_axis_name="s"),
    compiler_params=pltpu.CompilerParams(
        use_tc_tiling_on_sc=False, needs_layout_passes=False),
    scratch_types=[
        pltpu.VMEM((_NATTR, _CH), jnp.float32),
        pltpu.VMEM((_CH, _NATTR), jnp.float32),
    ],
)
def _yolo_sc(x_hbm, out_hbm, in_ref, out_ref):
  _sc_body(x_hbm, out_hbm, in_ref, out_ref)


def kernel(x):
  xr = x.reshape(_B, _NA * _NATTR, _GG)
  return _yolo_sc(xr)


# speedup vs baseline: 1.3714x; 1.3714x over previous
import functools

import jax
import jax.numpy as jnp
from jax import lax
from jax.experimental import pallas as pl
from jax.experimental.pallas import tpu as pltpu
from jax.experimental.pallas import tpu_sc as plsc

_G = 52
_GG = _G * _G
_NA = 3
_NATTR = 85
_B = 32
_STRIDE = 8.0
_ANCH = ((10.0, 13.0), (16.0, 30.0), (33.0, 23.0))
_CH = 208
_NCHUNK = _GG // _CH
_NV = _CH // 16
_NC = 2
_NS = 16


def _sigmoid(v):
  return 1.0 / (1.0 + jnp.exp(-v))


def _sc_body(x_hbm, out_hbm, in_ref, out_ref):
  b = lax.axis_index("s") * _NC + lax.axis_index("c")
  lane = lax.broadcasted_iota(jnp.int32, (16,), 0)

  for a in range(_NA):
    aw = _ANCH[a][0] / _STRIDE
    ah = _ANCH[a][1] / _STRIDE

    def chunk_body(k, carry, a=a, aw=aw, ah=ah):
      pltpu.sync_copy(
          x_hbm.at[b, pl.ds(a * _NATTR, _NATTR), pl.ds(k * _CH, _CH)], in_ref)

      @plsc.parallel_loop(0, _NV)
      def _box_body(j):
        sl = pl.ds(j * 16, 16)
        s_vec = lane + j * 16
        sg = k * _CH + s_vec
        q = lax.div(sg, _G)
        r = sg - q * _G
        gx = r.astype(jnp.float32)
        gy = q.astype(jnp.float32)
        r0 = (_sigmoid(in_ref[0, sl]) + gx) * _STRIDE
        r1 = (_sigmoid(in_ref[1, sl]) + gy) * _STRIDE
        r2 = (jnp.exp(in_ref[2, sl]) * aw) * _STRIDE
        r3 = (jnp.exp(in_ref[3, sl]) * ah) * _STRIDE
        for c, val in ((0, r0), (1, r1), (2, r2), (3, r3)):
          cv = jnp.full((16,), c, jnp.int32)
          plsc.store_scatter(out_ref, [s_vec, cv], val)

      @plsc.parallel_loop(4, _NATTR, unroll=2)
      def _c_body(c):
        cv = jnp.zeros((16,), jnp.int32) + c
        for j in range(_NV):
          val = _sigmoid(in_ref[c, pl.ds(j * 16, 16)])
          plsc.store_scatter(out_ref, [lane + j * 16, cv], val)

      pltpu.sync_copy(
          out_ref, out_hbm.at[b, pl.ds(a * _GG + k * _CH, _CH), :])
      return carry

    lax.fori_loop(0, _NCHUNK, chunk_body, 0)


@functools.partial(
    pl.kernel,
    out_type=jax.ShapeDtypeStruct((_B, _NA * _GG, _NATTR), jnp.float32),
    mesh=plsc.VectorSubcoreMesh(core_axis_name="c", subcore_axis_name="s"),
    compiler_params=pltpu.CompilerParams(
        use_tc_tiling_on_sc=False, needs_layout_passes=False),
    scratch_types=[
        pltpu.VMEM((_NATTR, _CH), jnp.float32),
        pltpu.VMEM((_CH, _NATTR), jnp.float32),
    ],
)
def _yolo_sc(x_hbm, out_hbm, in_ref, out_ref):
  _sc_body(x_hbm, out_hbm, in_ref, out_ref)


def kernel(x):
  xr = x.reshape(_B, _NA * _NATTR, _GG)
  return _yolo_sc(xr)
